# baseline (speedup 1.0000x reference)
import functools

import jax
import jax.numpy as jnp
from jax import lax
from jax.experimental import pallas as pl
from jax.experimental.pallas import tpu as pltpu
from jax.experimental.pallas import tpu_sc as plsc

_HIDDEN = 32
_GAMMA = 12.0
_BATCH = 16384
_NENT = 1000000

_INFO = plsc.get_sparse_core_info()
_NC = _INFO.num_cores
_NS = _INFO.num_subcores
_NW = _NC * _NS
_PER_W = _BATCH // _NW

_RANGE = 31360
_CW = 1024
_NCH = 32
_CMAX = 999040
_LCAP = 1040
_BCAP = 64
_RING = 8
_STAGE = 3 * _BATCH + 8
_DUMP = 3 * _BATCH


def _gather_body(hidx_hbm, ridx_hbm, tidx_hbm, entT, relT, valT, staged,
                 idxbuf, eh_l, ph_l, er_l, pr_l, et_l, pt_l,
                 bkt, bcnt, bufA, bufB, colbuf, semS, semW):
    w = lax.axis_index("s") * _NC + lax.axis_index("c")
    lo = w * _RANGE
    hi = lo + _RANGE + 384

    lanes = lax.iota(jnp.int32, 16)
    dlo = lax.iota(jnp.int32, 16)

    def scan_table(idx_hbm, elist, plist):
        pltpu.sync_copy(idx_hbm, idxbuf)

        def blk(b, cnt):
            for j in range(16):
                q = b * 16 + j
                v = idxbuf[pl.ds(q * 16, 16)]
                m = (v >= lo) & (v < hi)
                pos = q * 16 + lanes
                slots = cnt + plsc.cumsum(m.astype(jnp.int32)) - 1
                plsc.store_scatter(elist, [slots], v, mask=m)
                plsc.store_scatter(plist, [slots], pos, mask=m)
                pc = plsc.all_reduce_population_count(m)
                cnt = cnt + pc[0]
            return cnt
        return lax.fori_loop(0, 64, blk, jnp.int32(0))

    cnt_h = scan_table(hidx_hbm, eh_l, ph_l)
    cnt_r = scan_table(ridx_hbm, er_l, pr_l)
    cnt_t = scan_table(tidx_hbm, et_l, pt_l)

    def c_start(ci):
        return pl.multiple_of(jnp.minimum(lo + ci * _CW, _CMAX), 128)

    def bucketize(elist, plist, cnt, tno):
        zero = jnp.zeros((16,), jnp.int32)
        for q in range(2):
            bcnt[pl.ds((tno * 32 + q * 16) * 1, 16)] = zero

        def ent(i, _):
            li = jnp.full((16,), i, jnp.int32)
            e = plsc.load_gather(elist, [li])[0]
            pos = plsc.load_gather(plist, [li])[0]
            ci = jnp.minimum((e - lo) // _CW, _NCH - 1)
            b = tno * 32 + ci
            lb = jnp.full((16,), b, jnp.int32)
            c = plsc.load_gather(bcnt, [lb])[0]
            eloc = e - c_start(ci)
            packed = pos * 2048 + eloc
            l0 = lanes == 0
            plsc.store_scatter(bkt, [jnp.full((16,), b * _BCAP + c, jnp.int32)],
                               jnp.full((16,), packed, jnp.int32), mask=l0)
            plsc.store_scatter(bcnt, [lb],
                               jnp.full((16,), c + 1, jnp.int32), mask=l0)
            return ()

        lax.fori_loop(0, cnt, ent, ())

    bucketize(eh_l, ph_l, cnt_h, 0)
    bucketize(er_l, pr_l, cnt_r, 1)
    bucketize(et_l, pt_l, cnt_t, 2)

    def run_table(tab, tno, t_off):
        def issue(ci, buf):
            c0 = c_start(ci)
            for j in range(_CW // 128):
                pltpu.async_copy(tab.at[:, pl.ds(c0 + j * 128, 128)],
                                 buf.at[pl.ds(j * _HIDDEN, _HIDDEN), :], semS)

        def process(buf, ci):
            b = tno * 32 + ci
            lb = jnp.full((16,), b, jnp.int32)
            ccnt = plsc.load_gather(bcnt, [lb])[0]

            def pair(carry):
                i, o = carry

                @pl.when(o >= _RING)
                def _():
                    pltpu.make_async_copy(colbuf.at[0],
                                          staged.at[pl.ds(_DUMP * 32, 32)],
                                          semW).wait()

                o = jnp.where(o >= _RING, o - 1, o)
                li = jnp.full((16,), b * _BCAP + i, jnp.int32)
                packed = plsc.load_gather(bkt, [li])[0]
                eloc = packed % 2048
                pos = packed // 2048
                tb = (eloc // 128) * _HIDDEN
                le = jnp.full((16,), eloc % 128, jnp.int32)
                v0 = plsc.load_gather(buf, [tb + dlo, le])
                v1 = plsc.load_gather(buf, [tb + 16 + dlo, le])
                sl = i % _RING
                colbuf[sl, pl.ds(0, 16)] = v0
                colbuf[sl, pl.ds(16, 16)] = v1
                pltpu.async_copy(colbuf.at[sl],
                                 staged.at[pl.ds((t_off + pos) * 32, 32)],
                                 semW)
                return i + 1, o + 1

            _, o = lax.while_loop(lambda c: c[0] < ccnt, pair,
                                  (jnp.int32(0), jnp.int32(0)))

            def drain(o):
                pltpu.make_async_copy(colbuf.at[0],
                                      staged.at[pl.ds(_DUMP * 32, 32)],
                                      semW).wait()
                return o - 1

            lax.while_loop(lambda o: o > 0, drain, o)

        issue(0, bufA)

        def chunk_pair(m, _):
            ci0 = 2 * m
            ci1 = 2 * m + 1
            issue(ci1, bufB)
            pltpu.make_async_copy(tab.at[:, pl.ds(0, _CW)],
                                  bufA.at[pl.ds(0, _CW // 4), :], semS).wait()
            process(bufA, ci0)

            @pl.when(m < _NCH // 2 - 1)
            def _():
                issue(ci1 + 1, bufA)

            pltpu.make_async_copy(tab.at[:, pl.ds(0, _CW)],
                                  bufB.at[pl.ds(0, _CW // 4), :], semS).wait()
            process(bufB, ci1)
            return ()

        lax.fori_loop(0, _NCH // 2, chunk_pair, ())

    run_table(entT, 0, 0)
    run_table(relT, 1, _BATCH)
    run_table(valT, 2, 2 * _BATCH)


def _score_body(staged, out_hbm, h_v, r_v, t_v, o_v):
    wid = lax.axis_index("s") * _NC + lax.axis_index("c")
    lanes = lax.iota(jnp.int32, 16)
    base = wid * _PER_W * 32
    pltpu.sync_copy(staged.at[pl.ds(base, _PER_W * 32)], h_v)
    pltpu.sync_copy(staged.at[pl.ds(_BATCH * 32 + base, _PER_W * 32)], r_v)
    pltpu.sync_copy(staged.at[pl.ds(2 * _BATCH * 32 + base, _PER_W * 32)], t_v)

    def score_rows(i, _):
        acc = jnp.zeros((16,), jnp.float32)
        for k in range(16):
            lo = pl.ds((i * 16 + k) * 32, 16)
            hi = pl.ds((i * 16 + k) * 32 + 16, 16)
            a = jnp.abs(h_v[lo] + r_v[lo] - t_v[lo])
            b = jnp.abs(h_v[hi] + r_v[hi] - t_v[hi])
            acc = jnp.where(lanes == k, _GAMMA - jnp.sum(a + b), acc)
        o_v[pl.ds(i * 16, 16)] = acc
        return ()

    lax.fori_loop(0, _PER_W // 16, score_rows, ())
    pltpu.sync_copy(o_v, out_hbm.at[pl.ds(wid * _PER_W, _PER_W)])


@jax.jit
def _sc_score(hidx, ridx, tidx, entT, relT, valT):
    mesh = plsc.VectorSubcoreMesh(core_axis_name="c", subcore_axis_name="s")
    params = pltpu.CompilerParams(needs_layout_passes=False)
    gather = functools.partial(
        pl.kernel,
        mesh=mesh,
        compiler_params=params,
        out_type=jax.ShapeDtypeStruct((_STAGE * _HIDDEN,), jnp.float32),
        scratch_types=[
            pltpu.VMEM((_BATCH,), jnp.int32),
            pltpu.VMEM((_LCAP,), jnp.int32),
            pltpu.VMEM((_LCAP,), jnp.int32),
            pltpu.VMEM((_LCAP,), jnp.int32),
            pltpu.VMEM((_LCAP,), jnp.int32),
            pltpu.VMEM((_LCAP,), jnp.int32),
            pltpu.VMEM((_LCAP,), jnp.int32),
            pltpu.VMEM((3 * 32 * _BCAP,), jnp.int32),
            pltpu.VMEM((3 * 32,), jnp.int32),
            pltpu.VMEM((_CW // 128 * _HIDDEN, 128), jnp.float32),
            pltpu.VMEM((_CW // 128 * _HIDDEN, 128), jnp.float32),
            pltpu.VMEM((_RING, _HIDDEN), jnp.float32),
            pltpu.SemaphoreType.DMA,
            pltpu.SemaphoreType.DMA,
        ],
    )(_gather_body)
    staged = gather(hidx, ridx, tidx, entT, relT, valT)

    score = functools.partial(
        pl.kernel,
        mesh=mesh,
        compiler_params=params,
        out_type=jax.ShapeDtypeStruct((_BATCH,), jnp.float32),
        scratch_types=[
            pltpu.VMEM((_PER_W * _HIDDEN,), jnp.float32),
            pltpu.VMEM((_PER_W * _HIDDEN,), jnp.float32),
            pltpu.VMEM((_PER_W * _HIDDEN,), jnp.float32),
            pltpu.VMEM((_PER_W,), jnp.float32),
        ],
    )(_score_body)
    return score(staged)


def kernel(sample, entity_embedding, relation_embedding, value_embedding):
    idx = sample.astype(jnp.int32).T
    score = _sc_score(idx[0], idx[1], idx[2], entity_embedding.T,
                      relation_embedding.T, value_embedding.T)
    return score.reshape(_BATCH, 1)


# speedup vs baseline: 1.0502x; 1.0502x over previous
import functools

import jax
import jax.numpy as jnp
from jax import lax
from jax.experimental import pallas as pl
from jax.experimental.pallas import tpu as pltpu
from jax.experimental.pallas import tpu_sc as plsc

_HIDDEN = 32
_GAMMA = 12.0
_BATCH = 16384
_NENT = 1000000

_INFO = plsc.get_sparse_core_info()
_NC = _INFO.num_cores
_NS = _INFO.num_subcores
_NW = _NC * _NS
_PER_W = _BATCH // _NW

_RANGE = 31360
_CW = 512
_NCH = 64
_CMAX = 999552
_LCAP = 1040
_BCAP = 48
_RING = 8
_STAGE = 3 * _BATCH + 8
_DUMP = 3 * _BATCH


def _gather_body(hidx_hbm, ridx_hbm, tidx_hbm, entT, relT, valT, staged,
                 idxbuf, eh_l, ph_l, er_l, pr_l, et_l, pt_l,
                 bkt, bcnt, bufA, bufB, bufC, bufD, colbuf, semS, semW):
    w = lax.axis_index("s") * _NC + lax.axis_index("c")
    lo = w * _RANGE
    hi = lo + _RANGE + 384

    lanes = lax.iota(jnp.int32, 16)
    dlo = lax.iota(jnp.int32, 16)

    def scan_table(idx_hbm, elist, plist):
        pltpu.sync_copy(idx_hbm, idxbuf)

        def blk(b, cnt):
            for j in range(16):
                q = b * 16 + j
                v = idxbuf[pl.ds(q * 16, 16)]
                m = (v >= lo) & (v < hi)
                pos = q * 16 + lanes
                slots = cnt + plsc.cumsum(m.astype(jnp.int32)) - 1
                plsc.store_scatter(elist, [slots], v, mask=m)
                plsc.store_scatter(plist, [slots], pos, mask=m)
                pc = plsc.all_reduce_population_count(m)
                cnt = cnt + pc[0]
            return cnt
        return lax.fori_loop(0, 64, blk, jnp.int32(0))

    cnt_h = scan_table(hidx_hbm, eh_l, ph_l)
    cnt_r = scan_table(ridx_hbm, er_l, pr_l)
    cnt_t = scan_table(tidx_hbm, et_l, pt_l)

    def c_start(ci):
        return pl.multiple_of(jnp.minimum(lo + ci * _CW, _CMAX), 128)

    def bucketize(elist, plist, cnt, tno):
        zero = jnp.zeros((16,), jnp.int32)
        for q in range(4):
            bcnt[pl.ds(tno * _NCH + q * 16, 16)] = zero

        def ent(i, _):
            li = jnp.full((16,), i, jnp.int32)
            e = plsc.load_gather(elist, [li])[0]
            pos = plsc.load_gather(plist, [li])[0]
            ci = jnp.minimum((e - lo) // _CW, _NCH - 1)
            b = tno * _NCH + ci
            lb = jnp.full((16,), b, jnp.int32)
            c = plsc.load_gather(bcnt, [lb])[0]
            eloc = e - c_start(ci)
            packed = pos * 2048 + eloc
            l0 = lanes == 0
            plsc.store_scatter(bkt, [jnp.full((16,), b * _BCAP + c, jnp.int32)],
                               jnp.full((16,), packed, jnp.int32), mask=l0)
            plsc.store_scatter(bcnt, [lb],
                               jnp.full((16,), c + 1, jnp.int32), mask=l0)
            return ()

        lax.fori_loop(0, cnt, ent, ())

    bucketize(eh_l, ph_l, cnt_h, 0)
    bucketize(er_l, pr_l, cnt_r, 1)
    bucketize(et_l, pt_l, cnt_t, 2)

    def run_table(tab, tno, t_off):
        def issue(ci, buf):
            c0 = c_start(ci)
            for j in range(_CW // 128):
                pltpu.async_copy(tab.at[:, pl.ds(c0 + j * 128, 128)],
                                 buf.at[pl.ds(j * _HIDDEN, _HIDDEN), :], semS)

        def process(buf, ci):
            b = tno * _NCH + ci
            lb = jnp.full((16,), b, jnp.int32)
            ccnt = plsc.load_gather(bcnt, [lb])[0]

            def pair(carry):
                i, o = carry

                @pl.when(o >= _RING)
                def _():
                    pltpu.make_async_copy(colbuf.at[0],
                                          staged.at[pl.ds(_DUMP * 32, 32)],
                                          semW).wait()

                o = jnp.where(o >= _RING, o - 1, o)
                li = jnp.full((16,), b * _BCAP + i, jnp.int32)
                packed = plsc.load_gather(bkt, [li])[0]
                eloc = packed % 2048
                pos = packed // 2048
                tb = (eloc // 128) * _HIDDEN
                le = jnp.full((16,), eloc % 128, jnp.int32)
                v0 = plsc.load_gather(buf, [tb + dlo, le])
                v1 = plsc.load_gather(buf, [tb + 16 + dlo, le])
                sl = i % _RING
                colbuf[sl, pl.ds(0, 16)] = v0
                colbuf[sl, pl.ds(16, 16)] = v1
                pltpu.async_copy(colbuf.at[sl],
                                 staged.at[pl.ds((t_off + pos) * 32, 32)],
                                 semW)
                return i + 1, o + 1

            _, o = lax.while_loop(lambda c: c[0] < ccnt, pair,
                                  (jnp.int32(0), jnp.int32(0)))

            def drain(o):
                pltpu.make_async_copy(colbuf.at[0],
                                      staged.at[pl.ds(_DUMP * 32, 32)],
                                      semW).wait()
                return o - 1

            lax.while_loop(lambda o: o > 0, drain, o)

        bufs = [bufA, bufB, bufC, bufD]
        nrow = _CW // 128 * _HIDDEN
        for s in range(3):
            issue(s, bufs[s])

        def quad(m, _):
            for s in range(4):
                @pl.when(4 * m + s + 3 < _NCH)
                def _(s=s):
                    issue(4 * m + s + 3, bufs[(s + 3) % 4])

                pltpu.make_async_copy(tab.at[:, pl.ds(0, _CW)],
                                      bufs[s].at[pl.ds(0, nrow), :],
                                      semS).wait()
                process(bufs[s], 4 * m + s)
            return ()

        lax.fori_loop(0, _NCH // 4, quad, ())

    run_table(entT, 0, 0)
    run_table(relT, 1, _BATCH)
    run_table(valT, 2, 2 * _BATCH)


def _score_body(staged, out_hbm, h_v, r_v, t_v, o_v):
    wid = lax.axis_index("s") * _NC + lax.axis_index("c")
    lanes = lax.iota(jnp.int32, 16)
    base = wid * _PER_W * 32
    pltpu.sync_copy(staged.at[pl.ds(base, _PER_W * 32)], h_v)
    pltpu.sync_copy(staged.at[pl.ds(_BATCH * 32 + base, _PER_W * 32)], r_v)
    pltpu.sync_copy(staged.at[pl.ds(2 * _BATCH * 32 + base, _PER_W * 32)], t_v)

    def score_rows(i, _):
        acc = jnp.zeros((16,), jnp.float32)
        for k in range(16):
            lo = pl.ds((i * 16 + k) * 32, 16)
            hi = pl.ds((i * 16 + k) * 32 + 16, 16)
            a = jnp.abs(h_v[lo] + r_v[lo] - t_v[lo])
            b = jnp.abs(h_v[hi] + r_v[hi] - t_v[hi])
            acc = jnp.where(lanes == k, _GAMMA - jnp.sum(a + b), acc)
        o_v[pl.ds(i * 16, 16)] = acc
        return ()

    lax.fori_loop(0, _PER_W // 16, score_rows, ())
    pltpu.sync_copy(o_v, out_hbm.at[pl.ds(wid * _PER_W, _PER_W)])


@jax.jit
def _sc_score(hidx, ridx, tidx, entT, relT, valT):
    mesh = plsc.VectorSubcoreMesh(core_axis_name="c", subcore_axis_name="s")
    params = pltpu.CompilerParams(needs_layout_passes=False)
    gather = functools.partial(
        pl.kernel,
        mesh=mesh,
        compiler_params=params,
        out_type=jax.ShapeDtypeStruct((_STAGE * _HIDDEN,), jnp.float32),
        scratch_types=[
            pltpu.VMEM((_BATCH,), jnp.int32),
            pltpu.VMEM((_LCAP,), jnp.int32),
            pltpu.VMEM((_LCAP,), jnp.int32),
            pltpu.VMEM((_LCAP,), jnp.int32),
            pltpu.VMEM((_LCAP,), jnp.int32),
            pltpu.VMEM((_LCAP,), jnp.int32),
            pltpu.VMEM((_LCAP,), jnp.int32),
            pltpu.VMEM((3 * _NCH * _BCAP,), jnp.int32),
            pltpu.VMEM((3 * _NCH,), jnp.int32),
            pltpu.VMEM((_CW // 128 * _HIDDEN, 128), jnp.float32),
            pltpu.VMEM((_CW // 128 * _HIDDEN, 128), jnp.float32),
            pltpu.VMEM((_CW // 128 * _HIDDEN, 128), jnp.float32),
            pltpu.VMEM((_CW // 128 * _HIDDEN, 128), jnp.float32),
            pltpu.VMEM((_RING, _HIDDEN), jnp.float32),
            pltpu.SemaphoreType.DMA,
            pltpu.SemaphoreType.DMA,
        ],
    )(_gather_body)
    staged = gather(hidx, ridx, tidx, entT, relT, valT)

    score = functools.partial(
        pl.kernel,
        mesh=mesh,
        compiler_params=params,
        out_type=jax.ShapeDtypeStruct((_BATCH,), jnp.float32),
        scratch_types=[
            pltpu.VMEM((_PER_W * _HIDDEN,), jnp.float32),
            pltpu.VMEM((_PER_W * _HIDDEN,), jnp.float32),
            pltpu.VMEM((_PER_W * _HIDDEN,), jnp.float32),
            pltpu.VMEM((_PER_W,), jnp.float32),
        ],
    )(_score_body)
    return score(staged)


def kernel(sample, entity_embedding, relation_embedding, value_embedding):
    idx = sample.astype(jnp.int32).T
    score = _sc_score(idx[0], idx[1], idx[2], entity_embedding.T,
                      relation_embedding.T, value_embedding.T)
    return score.reshape(_BATCH, 1)


# speedup vs baseline: 1.1584x; 1.1030x over previous
import functools

import jax
import jax.numpy as jnp
from jax import lax
from jax.experimental import pallas as pl
from jax.experimental.pallas import tpu as pltpu
from jax.experimental.pallas import tpu_sc as plsc

_HIDDEN = 32
_GAMMA = 12.0
_BATCH = 16384
_NENT = 1000000

_INFO = plsc.get_sparse_core_info()
_NC = _INFO.num_cores
_NS = _INFO.num_subcores
_NW = _NC * _NS
_PER_W = _BATCH // _NW

_RANGE = 31360
_CW = 1024
_NCH = 32
_CMAX = 999040
_LCAP = 1040
_CCAP = 144
_RING = 8
_STAGE = 3 * _BATCH + 8
_DUMP = 3 * _BATCH


def _gather_body(hidx_hbm, ridx_hbm, tidx_hbm, entT, relT, valT, staged,
                 idxbuf, eh_l, ph_l, er_l, pr_l, et_l, pt_l,
                 celoc, cpos, bufA, bufB, colbuf, semS, semW):
    w = lax.axis_index("s") * _NC + lax.axis_index("c")
    lo = w * _RANGE
    hi = lo + _RANGE + 384

    lanes = lax.iota(jnp.int32, 16)
    dlo = lax.iota(jnp.int32, 16)

    def scan_table(idx_hbm, elist, plist):
        pltpu.sync_copy(idx_hbm, idxbuf)

        def blk(b, cnt):
            for j in range(16):
                q = b * 16 + j
                v = idxbuf[pl.ds(q * 16, 16)]
                m = (v >= lo) & (v < hi)
                pos = q * 16 + lanes
                slots = cnt + plsc.cumsum(m.astype(jnp.int32)) - 1
                plsc.store_scatter(elist, [slots], v, mask=m)
                plsc.store_scatter(plist, [slots], pos, mask=m)
                pc = plsc.all_reduce_population_count(m)
                cnt = cnt + pc[0]
            return cnt
        return lax.fori_loop(0, 64, blk, jnp.int32(0))

    cnt_h = scan_table(hidx_hbm, eh_l, ph_l)
    cnt_r = scan_table(ridx_hbm, er_l, pr_l)
    cnt_t = scan_table(tidx_hbm, et_l, pt_l)

    def c_start(ci):
        return pl.multiple_of(jnp.minimum(lo + ci * _CW, _CMAX), 128)

    def run_table(tab, elist, plist, cnt, t_off):
        def issue(ci, buf):
            c0 = c_start(ci)
            for j in range(_CW // 128):
                pltpu.async_copy(tab.at[:, pl.ds(c0 + j * 128, 128)],
                                 buf.at[pl.ds(j * _HIDDEN, _HIDDEN), :], semS)

        def process(buf, c0, width):
            def sel(carry):
                vi, ccnt = carry
                ev = elist[pl.ds(vi, 16)]
                pv = plist[pl.ds(vi, 16)]
                m = (ev >= c0) & (ev < c0 + width) & ((vi + lanes) < cnt)
                slots = ccnt + plsc.cumsum(m.astype(jnp.int32)) - 1
                plsc.store_scatter(celoc, [slots], ev - c0, mask=m)
                plsc.store_scatter(cpos, [slots], pv, mask=m)
                pc = plsc.all_reduce_population_count(m)
                return vi + 16, ccnt + pc[0]

            _, ccnt = lax.while_loop(lambda c: c[0] < cnt, sel,
                                     (jnp.int32(0), jnp.int32(0)))

            def pair(carry):
                i, o = carry

                @pl.when(o >= _RING)
                def _():
                    pltpu.make_async_copy(colbuf.at[0],
                                          staged.at[pl.ds(_DUMP * 32, 32)],
                                          semW).wait()

                o = jnp.where(o >= _RING, o - 1, o)
                li = jnp.full((16,), i, jnp.int32)
                eloc = plsc.load_gather(celoc, [li])[0]
                pos = plsc.load_gather(cpos, [li])[0]
                tb = (eloc // 128) * _HIDDEN
                le = jnp.full((16,), eloc % 128, jnp.int32)
                v0 = plsc.load_gather(buf, [tb + dlo, le])
                v1 = plsc.load_gather(buf, [tb + 16 + dlo, le])
                sl = i % _RING
                colbuf[sl, pl.ds(0, 16)] = v0
                colbuf[sl, pl.ds(16, 16)] = v1
                pltpu.async_copy(colbuf.at[sl],
                                 staged.at[pl.ds((t_off + pos) * 32, 32)],
                                 semW)
                return i + 1, o + 1

            _, o = lax.while_loop(lambda c: c[0] < ccnt, pair,
                                  (jnp.int32(0), jnp.int32(0)))

            def drain(o):
                pltpu.make_async_copy(colbuf.at[0],
                                      staged.at[pl.ds(_DUMP * 32, 32)],
                                      semW).wait()
                return o - 1

            lax.while_loop(lambda o: o > 0, drain, o)

        issue(0, bufA)

        def chunk_pair(m, _):
            ci0 = 2 * m
            ci1 = 2 * m + 1
            issue(ci1, bufB)
            pltpu.make_async_copy(tab.at[:, pl.ds(0, _CW)],
                                  bufA.at[pl.ds(0, _CW // 4), :], semS).wait()
            process(bufA, c_start(ci0), _CW)

            @pl.when(m < _NCH // 2 - 1)
            def _():
                issue(ci1 + 1, bufA)

            pltpu.make_async_copy(tab.at[:, pl.ds(0, _CW)],
                                  bufB.at[pl.ds(0, _CW // 4), :], semS).wait()
            process(bufB, c_start(ci1), _CW)
            return ()

        lax.fori_loop(0, _NCH // 2, chunk_pair, ())

    run_table(entT, eh_l, ph_l, cnt_h, 0)
    run_table(relT, er_l, pr_l, cnt_r, _BATCH)
    run_table(valT, et_l, pt_l, cnt_t, 2 * _BATCH)


def _score_body(staged, out_hbm, h_v, r_v, t_v, o_v):
    wid = lax.axis_index("s") * _NC + lax.axis_index("c")
    lanes = lax.iota(jnp.int32, 16)
    base = wid * _PER_W * 32
    pltpu.sync_copy(staged.at[pl.ds(base, _PER_W * 32)], h_v)
    pltpu.sync_copy(staged.at[pl.ds(_BATCH * 32 + base, _PER_W * 32)], r_v)
    pltpu.sync_copy(staged.at[pl.ds(2 * _BATCH * 32 + base, _PER_W * 32)], t_v)

    def score_rows(i, _):
        acc = jnp.zeros((16,), jnp.float32)
        for k in range(16):
            lo = pl.ds((i * 16 + k) * 32, 16)
            hi = pl.ds((i * 16 + k) * 32 + 16, 16)
            a = jnp.abs(h_v[lo] + r_v[lo] - t_v[lo])
            b = jnp.abs(h_v[hi] + r_v[hi] - t_v[hi])
            acc = jnp.where(lanes == k, _GAMMA - jnp.sum(a + b), acc)
        o_v[pl.ds(i * 16, 16)] = acc
        return ()

    lax.fori_loop(0, _PER_W // 16, score_rows, ())
    pltpu.sync_copy(o_v, out_hbm.at[pl.ds(wid * _PER_W, _PER_W)])


@jax.jit
def _sc_score(hidx, ridx, tidx, entT, relT, valT):
    mesh = plsc.VectorSubcoreMesh(core_axis_name="c", subcore_axis_name="s")
    params = pltpu.CompilerParams(needs_layout_passes=False)
    gather = functools.partial(
        pl.kernel,
        mesh=mesh,
        compiler_params=params,
        out_type=jax.ShapeDtypeStruct((_STAGE * _HIDDEN,), jnp.float32),
        scratch_types=[
            pltpu.VMEM((_BATCH,), jnp.int32),
            pltpu.VMEM((_LCAP,), jnp.int32),
            pltpu.VMEM((_LCAP,), jnp.int32),
            pltpu.VMEM((_LCAP,), jnp.int32),
            pltpu.VMEM((_LCAP,), jnp.int32),
            pltpu.VMEM((_LCAP,), jnp.int32),
            pltpu.VMEM((_LCAP,), jnp.int32),
            pltpu.VMEM((_CCAP,), jnp.int32),
            pltpu.VMEM((_CCAP,), jnp.int32),
            pltpu.VMEM((_CW // 128 * _HIDDEN, 128), jnp.float32),
            pltpu.VMEM((_CW // 128 * _HIDDEN, 128), jnp.float32),
            pltpu.VMEM((_RING, _HIDDEN), jnp.float32),
            pltpu.SemaphoreType.DMA,
            pltpu.SemaphoreType.DMA,
        ],
    )(_gather_body)
    staged = gather(hidx, ridx, tidx, entT, relT, valT)

    score = functools.partial(
        pl.kernel,
        mesh=mesh,
        compiler_params=params,
        out_type=jax.ShapeDtypeStruct((_BATCH,), jnp.float32),
        scratch_types=[
            pltpu.VMEM((_PER_W * _HIDDEN,), jnp.float32),
            pltpu.VMEM((_PER_W * _HIDDEN,), jnp.float32),
            pltpu.VMEM((_PER_W * _HIDDEN,), jnp.float32),
            pltpu.VMEM((_PER_W,), jnp.float32),
        ],
    )(_score_body)
    return score(staged)


def kernel(sample, entity_embedding, relation_embedding, value_embedding):
    idx = sample.astype(jnp.int32).T
    score = _sc_score(idx[0], idx[1], idx[2], entity_embedding.T,
                      relation_embedding.T, value_embedding.T)
    return score.reshape(_BATCH, 1)
